# baseline (speedup 1.0000x reference)
import jax
import jax.numpy as jnp
from jax.experimental import pallas as pl


_ROWS = 20480
_BLOCK_ROWS = 2560


def _wav_copy_kernel(wav_ref, wav_out_ref):
    wav_out_ref[...] = wav_ref[...]


def _len_copy_kernel(len_ref, len_out_ref):
    len_out_ref[...] = len_ref[...]


def kernel(wav_batch, lengths):
    wav2d = wav_batch.reshape(_ROWS, 128)
    wav_out = pl.pallas_call(
        _wav_copy_kernel,
        grid=(_ROWS // _BLOCK_ROWS,),
        in_specs=[pl.BlockSpec((_BLOCK_ROWS, 128), lambda i: (i, 0))],
        out_specs=pl.BlockSpec((_BLOCK_ROWS, 128), lambda i: (i, 0)),
        out_shape=jax.ShapeDtypeStruct((_ROWS, 128), wav_batch.dtype),
    )(wav2d)

    lengths_2d = jnp.asarray(lengths).astype(jnp.int32).reshape(1, lengths.shape[0])
    len_out = pl.pallas_call(
        _len_copy_kernel,
        out_shape=jax.ShapeDtypeStruct(lengths_2d.shape, jnp.int32),
    )(lengths_2d)

    return wav_out.reshape(wav_batch.shape), len_out.reshape(lengths.shape)


# speedup vs baseline: 1.0053x; 1.0053x over previous
import jax
import jax.numpy as jnp
from jax.experimental import pallas as pl


_ROWS = 20480
_BLOCK_ROWS = 2560


def _splayer_kernel(wav_ref, len_ref, wav_out_ref, len_out_ref):
    wav_out_ref[...] = wav_ref[...]
    len_out_ref[...] = len_ref[...]


def kernel(wav_batch, lengths):
    wav2d = wav_batch.reshape(_ROWS, 128)
    lengths_2d = jnp.asarray(lengths).astype(jnp.int32).reshape(1, lengths.shape[0])
    wav_out, len_out = pl.pallas_call(
        _splayer_kernel,
        grid=(_ROWS // _BLOCK_ROWS,),
        in_specs=[
            pl.BlockSpec((_BLOCK_ROWS, 128), lambda i: (i, 0)),
            pl.BlockSpec(lengths_2d.shape, lambda i: (0, 0)),
        ],
        out_specs=[
            pl.BlockSpec((_BLOCK_ROWS, 128), lambda i: (i, 0)),
            pl.BlockSpec(lengths_2d.shape, lambda i: (0, 0)),
        ],
        out_shape=[
            jax.ShapeDtypeStruct((_ROWS, 128), wav_batch.dtype),
            jax.ShapeDtypeStruct(lengths_2d.shape, jnp.int32),
        ],
    )(wav2d, lengths_2d)
    return wav_out.reshape(wav_batch.shape), len_out.reshape(lengths.shape)


# speedup vs baseline: 1.8527x; 1.8428x over previous
import jax
import jax.numpy as jnp
from jax.experimental import pallas as pl


def _splayer_kernel(wav_ref, len_ref, wav_out_ref, len_out_ref):
    wav_out_ref[...] = wav_ref[...]
    len_out_ref[...] = len_ref[...]


def kernel(wav_batch, lengths):
    b, t, f = wav_batch.shape
    lengths_2d = jnp.asarray(lengths).astype(jnp.int32).reshape(1, lengths.shape[0])
    wav_out, len_out = pl.pallas_call(
        _splayer_kernel,
        grid=(b,),
        in_specs=[
            pl.BlockSpec((1, t, f), lambda i: (i, 0, 0)),
            pl.BlockSpec(lengths_2d.shape, lambda i: (0, 0)),
        ],
        out_specs=[
            pl.BlockSpec((1, t, f), lambda i: (i, 0, 0)),
            pl.BlockSpec(lengths_2d.shape, lambda i: (0, 0)),
        ],
        out_shape=[
            jax.ShapeDtypeStruct(wav_batch.shape, wav_batch.dtype),
            jax.ShapeDtypeStruct(lengths_2d.shape, jnp.int32),
        ],
    )(wav_batch, lengths_2d)
    return wav_out, len_out.reshape(lengths.shape)
